# Initial kernel scaffold; baseline (speedup 1.0000x reference)
import jax
import jax.numpy as jnp
from jax.experimental import pallas as pl


def kernel(sr, hr):
    raise NotImplementedError("write your pallas kernel here")


# speedup vs baseline: 2887.6961x; 2887.6961x over previous
import functools

import jax
import jax.numpy as jnp
from jax import lax
from jax.experimental import pallas as pl
from jax.experimental.pallas import tpu as pltpu
from jax.experimental.pallas import tpu_sc as plsc

_N = 4096
_P = 512
_NP = 8
_NPATCH = 64
_R = 64
_NCHUNK = _P // _R
_NBIN = 256
_NPIX = _P * _P

_MESH = plsc.VectorSubcoreMesh(core_axis_name="c", subcore_axis_name="s")


@functools.partial(
    pl.kernel,
    out_type=(
        jax.ShapeDtypeStruct((_NPATCH, 16 * _NBIN), jnp.float32),
        jax.ShapeDtypeStruct((_NPATCH, 16), jnp.float32),
    ),
    mesh=_MESH,
    compiler_params=pltpu.CompilerParams(needs_layout_passes=False),
    scratch_types=[
        pltpu.VMEM((_R, _P), jnp.float32),
        pltpu.VMEM((_R, _P), jnp.float32),
        pltpu.VMEM((16 * _NBIN,), jnp.float32),
        pltpu.VMEM((16,), jnp.float32),
    ],
)
def _sc_stats(hr_hbm, sr_hbm, hist_out, asum_out, hr_v, sr_v, hist_v, asum_v):
    wid = lax.axis_index("s") * 2 + lax.axis_index("c")
    lane_base = lax.iota(jnp.int32, 16) * _NBIN
    ones = jnp.ones((16,), jnp.float32)
    zeros16 = jnp.zeros((16,), jnp.float32)

    for pp in range(2):
        p = wid * 2 + pp
        row0 = (p // _NP) * _P
        col0 = (p % _NP) * _P

        def zero_body(i, carry):
            hist_v[pl.ds(i * 16, 16)] = zeros16
            return carry

        lax.fori_loop(0, _NBIN, zero_body, 0)

        acc = jnp.zeros((16,), jnp.float32)
        for c in range(_NCHUNK):
            r0 = row0 + c * _R
            pltpu.sync_copy(hr_hbm.at[pl.ds(r0, _R), pl.ds(col0, _P)], hr_v)
            pltpu.sync_copy(sr_hbm.at[pl.ds(r0, _R), pl.ds(col0, _P)], sr_v)

            def row_body(r, a):
                def vec_body(k, a2):
                    h = hr_v[r, pl.ds(k * 16, 16)]
                    s = sr_v[r, pl.ds(k * 16, 16)]
                    b = (h * 255.0).astype(jnp.int32)
                    plsc.addupdate_scatter(hist_v, [lane_base + b], ones)
                    return a2 + jnp.abs(h - s)

                return lax.fori_loop(0, _P // 16, vec_body, a)

            acc = lax.fori_loop(0, _R, row_body, acc)

        asum_v[...] = acc
        pltpu.sync_copy(hist_v, hist_out.at[p])
        pltpu.sync_copy(asum_v, asum_out.at[p])


def _combine_body(hist_ref, asum_ref, out_ref):
    h = hist_ref[...]
    counts = jnp.sum(h, axis=1)
    prob = counts * (1.0 / _NPIX)
    pos = counts > 0.0
    logp = jnp.log(jnp.where(pos, prob, 1.0))
    terms = jnp.where(pos, prob * logp, 0.0) * (-1.0 / jnp.log(2.0))

    ent = jnp.sum(terms[:, 0:16], axis=1, keepdims=True)
    comp = jnp.zeros_like(ent)
    for g in range(1, 16):
        y = jnp.sum(terms[:, g * 16:(g + 1) * 16], axis=1, keepdims=True) - comp
        t = ent + y
        comp = (t - ent) - y
        ent = t

    emin = jnp.min(ent)
    emax = jnp.max(ent)
    w = (ent - emin) / emax
    s = jnp.sum(asum_ref[...], axis=1, keepdims=True)
    out_ref[...] = jnp.reshape(jnp.sum(w * s) * (1.0 / (_N * _N)), (1, 1))


def kernel(sr, hr):
    hist, asum = _sc_stats(hr, sr)
    out = pl.pallas_call(
        _combine_body,
        out_shape=jax.ShapeDtypeStruct((1, 1), jnp.float32),
    )(hist.reshape(_NPATCH, 16, _NBIN), asum)
    return out[0, 0]


# speedup vs baseline: 10734.5895x; 3.7174x over previous
import functools

import jax
import jax.numpy as jnp
from jax import lax
from jax.experimental import pallas as pl
from jax.experimental.pallas import tpu as pltpu
from jax.experimental.pallas import tpu_sc as plsc

_N = 4096
_P = 512
_NP = 8
_NPATCH = 64
_R = 32
_SLAB_PER_PATCH = _P // _R
_NBIN = 256
_NPIX = _P * _P

_MESH = plsc.VectorSubcoreMesh(core_axis_name="c", subcore_axis_name="s")


@functools.partial(
    pl.kernel,
    out_type=(
        jax.ShapeDtypeStruct((_NPATCH, 16 * _NBIN), jnp.float32),
        jax.ShapeDtypeStruct((_NPATCH, 16), jnp.float32),
    ),
    mesh=_MESH,
    compiler_params=pltpu.CompilerParams(needs_layout_passes=False),
    scratch_types=[
        pltpu.VMEM((2, _R, _P), jnp.float32),
        pltpu.VMEM((2, _R, _P), jnp.float32),
        pltpu.VMEM((16 * _NBIN,), jnp.float32),
        pltpu.VMEM((16,), jnp.float32),
        pltpu.SemaphoreType.DMA,
        pltpu.SemaphoreType.DMA,
        pltpu.SemaphoreType.DMA,
        pltpu.SemaphoreType.DMA,
    ],
)
def _sc_stats(hr_hbm, sr_hbm, hist_out, asum_out,
              hr_v, sr_v, hist_v, asum_v, semh0, semh1, sems0, sems1):
    wid = lax.axis_index("s") * 2 + lax.axis_index("c")
    lane_base = lax.iota(jnp.int32, 16) * _NBIN
    ones = jnp.ones((16,), jnp.float32)
    zeros16 = jnp.zeros((16,), jnp.float32)
    semh = (semh0, semh1)
    sems = (sems0, sems1)

    def zero_hist():
        def zb(i, carry):
            hist_v[pl.ds(i * 16, 16)] = zeros16
            return carry
        lax.fori_loop(0, _NBIN, zb, 0)

    zero_hist()

    _U = 8
    _FLAT = _R * (_P // 16)
    _VSH = 5
    _PAIRS = _SLAB_PER_PATCH // 2

    def patch_src(hbm, p, t):
        r0 = (p // _NP) * _P + t * _R
        c0 = (p % _NP) * _P
        return hbm.at[pl.ds(r0, _R), pl.ds(c0, _P)]

    def start_slab(p, t, b):
        pltpu.async_copy(patch_src(hr_hbm, p, t), hr_v.at[b], semh[b])
        pltpu.async_copy(patch_src(sr_hbm, p, t), sr_v.at[b], sems[b])

    def wait_slab(p, b):
        pltpu.make_async_copy(patch_src(hr_hbm, p, 0), hr_v.at[b], semh[b]).wait()
        pltpu.make_async_copy(patch_src(sr_hbm, p, 0), sr_v.at[b], sems[b]).wait()

    for pp in range(2):
        p = wid * 2 + pp
        start_slab(p, 0, 0)
        start_slab(p, 1, 1)
        accs = tuple(jnp.zeros((16,), jnp.float32) for _ in range(_U))

        def pair_body(j, accs, p=p):
            for b in range(2):
                t = 2 * j + b
                wait_slab(p, b)

                def slab_body(i, a, b=b):
                    out = []
                    for u in range(_U):
                        v = i + u
                        r = v >> _VSH
                        k = v - (r << _VSH)
                        h = hr_v[b, r, pl.ds(k * 16, 16)]
                        s = sr_v[b, r, pl.ds(k * 16, 16)]
                        bin_i = (h * 255.0).astype(jnp.int32)
                        plsc.addupdate_scatter(hist_v, [lane_base + bin_i], ones)
                        out.append(a[u] + jnp.abs(h - s))
                    return tuple(out)

                accs = plsc.parallel_loop(0, _FLAT, _U, carry=accs)(slab_body)

                @pl.when(t + 2 < _SLAB_PER_PATCH)
                def _(p=p, t=t, b=b):
                    start_slab(p, t + 2, b)
            return accs

        accs = lax.fori_loop(0, _PAIRS, pair_body, accs)
        a01 = accs[0] + accs[1]
        a23 = accs[2] + accs[3]
        a45 = accs[4] + accs[5]
        a67 = accs[6] + accs[7]
        asum_v[...] = (a01 + a23) + (a45 + a67)
        pltpu.sync_copy(asum_v, asum_out.at[p])
        pltpu.sync_copy(hist_v, hist_out.at[p])
        if pp == 0:
            zero_hist()


def _combine_body(hist_ref, asum_ref, out_ref):
    h = hist_ref[...]
    counts = jnp.sum(h, axis=1)
    prob = counts * (1.0 / _NPIX)
    pos = counts > 0.0
    logp = jnp.log(jnp.where(pos, prob, 1.0))
    terms = jnp.where(pos, prob * logp, 0.0) * (-1.0 / jnp.log(2.0))

    ent = jnp.sum(terms[:, 0:16], axis=1, keepdims=True)
    comp = jnp.zeros_like(ent)
    for g in range(1, 16):
        y = jnp.sum(terms[:, g * 16:(g + 1) * 16], axis=1, keepdims=True) - comp
        t = ent + y
        comp = (t - ent) - y
        ent = t

    emin = jnp.min(ent)
    emax = jnp.max(ent)
    w = (ent - emin) / emax
    s = jnp.sum(asum_ref[...], axis=1, keepdims=True)
    out_ref[...] = jnp.reshape(jnp.sum(w * s) * (1.0 / (_N * _N)), (1, 1))


def kernel(sr, hr):
    hist, asum = _sc_stats(hr, sr)
    out = pl.pallas_call(
        _combine_body,
        out_shape=jax.ShapeDtypeStruct((1, 1), jnp.float32),
    )(hist.reshape(_NPATCH, 16, _NBIN), asum)
    return out[0, 0]


# speedup vs baseline: 12528.3805x; 1.1671x over previous
import functools

import jax
import jax.numpy as jnp
from jax import lax
from jax.experimental import pallas as pl
from jax.experimental.pallas import tpu as pltpu
from jax.experimental.pallas import tpu_sc as plsc

_N = 4096
_P = 512
_NP = 8
_NPATCH = 64
_R = 64
_SLAB_PER_PATCH = _P // _R
_NSLAB = 2 * _SLAB_PER_PATCH
_NBIN = 256
_NPIX = _P * _P

_MESH = plsc.VectorSubcoreMesh(core_axis_name="c", subcore_axis_name="s")


@functools.partial(
    pl.kernel,
    out_type=jax.ShapeDtypeStruct((_NPATCH, 16 * _NBIN), jnp.float32),
    mesh=_MESH,
    compiler_params=pltpu.CompilerParams(needs_layout_passes=False),
    scratch_types=[
        pltpu.VMEM((2, _R, _P), jnp.float32),
        pltpu.VMEM((16 * _NBIN,), jnp.float32),
        pltpu.SemaphoreType.DMA,
        pltpu.SemaphoreType.DMA,
    ],
)
def _sc_hist(hr_hbm, hist_out, hr_v, hist_v, sem0, sem1):
    wid = lax.axis_index("s") * 2 + lax.axis_index("c")
    lane_base = lax.iota(jnp.int32, 16) * _NBIN
    ones = jnp.ones((16,), jnp.float32)
    zeros16 = jnp.zeros((16,), jnp.float32)
    sems = (sem0, sem1)

    def src(t):
        p = wid * 2 + t // _SLAB_PER_PATCH
        r0 = (p // _NP) * _P + (t % _SLAB_PER_PATCH) * _R
        c0 = (p % _NP) * _P
        return hr_hbm.at[pl.ds(r0, _R), pl.ds(c0, _P)]

    def zero_hist():
        def zb(i, carry):
            hist_v[pl.ds(i * 16, 16)] = zeros16
            return carry
        lax.fori_loop(0, _NBIN, zb, 0)

    zero_hist()
    _U = 8
    _FLAT = _R * (_P // 16)
    _VSH = 5
    _PAIRS = _SLAB_PER_PATCH // 2

    def patch_src(p, t):
        r0 = (p // _NP) * _P + t * _R
        c0 = (p % _NP) * _P
        return hr_hbm.at[pl.ds(r0, _R), pl.ds(c0, _P)]

    for pp in range(2):
        p = wid * 2 + pp
        pltpu.async_copy(patch_src(p, 0), hr_v.at[0], sems[0])
        pltpu.async_copy(patch_src(p, 1), hr_v.at[1], sems[1])

        def pair_body(j, carry, p=p):
            for b in range(2):
                t = 2 * j + b
                pltpu.make_async_copy(patch_src(p, 0), hr_v.at[b], sems[b]).wait()

                def slab_body(i, c, b=b):
                    for u in range(_U):
                        v = i + u
                        r = v >> _VSH
                        k = v - (r << _VSH)
                        h = hr_v[b, r, pl.ds(k * 16, 16)]
                        bin_i = (h * 255.0).astype(jnp.int32)
                        plsc.addupdate_scatter(hist_v, [lane_base + bin_i], ones)
                    return c

                plsc.parallel_loop(0, _FLAT, _U, carry=jnp.int32(0))(slab_body)

                @pl.when(t + 2 < _SLAB_PER_PATCH)
                def _(p=p, t=t, b=b):
                    pltpu.async_copy(patch_src(p, t + 2), hr_v.at[b], sems[b])
            return carry

        lax.fori_loop(0, _PAIRS, pair_body, 0)
        pltpu.sync_copy(hist_v, hist_out.at[p])
        if pp == 0:
            zero_hist()


def _abs_body(sr_ref, hr_ref, out_ref):
    d = jnp.abs(sr_ref[...] - hr_ref[...])
    out_ref[...] = jnp.sum(d, axis=0).reshape(1, 1, _P)


def _tc_abs(sr, hr):
    return pl.pallas_call(
        _abs_body,
        grid=(_NPATCH,),
        in_specs=[
            pl.BlockSpec((_P, _P), lambda p: (p // _NP, p % _NP)),
            pl.BlockSpec((_P, _P), lambda p: (p // _NP, p % _NP)),
        ],
        out_specs=pl.BlockSpec((1, 1, _P), lambda p: (p, 0, 0)),
        out_shape=jax.ShapeDtypeStruct((_NPATCH, 1, _P), jnp.float32),
    )(sr, hr)


def _combine_body(hist_ref, psum_ref, out_ref):
    h = hist_ref[...]
    counts = jnp.sum(h, axis=1)
    prob = counts * (1.0 / _NPIX)
    pos = counts > 0.0
    logp = jnp.log(jnp.where(pos, prob, 1.0))
    terms = jnp.where(pos, prob * logp, 0.0) * (-1.0 / jnp.log(2.0))

    ent = jnp.sum(terms[:, 0:16], axis=1, keepdims=True)
    comp = jnp.zeros_like(ent)
    for g in range(1, 16):
        y = jnp.sum(terms[:, g * 16:(g + 1) * 16], axis=1, keepdims=True) - comp
        t = ent + y
        comp = (t - ent) - y
        ent = t

    emin = jnp.min(ent)
    emax = jnp.max(ent)
    w = (ent - emin) / emax
    s = jnp.sum(psum_ref[...], axis=1, keepdims=True)
    out_ref[...] = jnp.reshape(jnp.sum(w * s) * (1.0 / (_N * _N)), (1, 1))


def kernel(sr, hr):
    hist = _sc_hist(hr)
    psum = _tc_abs(sr, hr)
    out = pl.pallas_call(
        _combine_body,
        out_shape=jax.ShapeDtypeStruct((1, 1), jnp.float32),
    )(hist.reshape(_NPATCH, 16, _NBIN), psum.reshape(_NPATCH, _P))
    return out[0, 0]
